# baseline (speedup 1.0000x reference)
import functools

import jax
import jax.numpy as jnp
from jax import lax
from jax.experimental import pallas as pl
from jax.experimental.pallas import tpu as pltpu
from jax.experimental.pallas import tpu_sc as plsc

N_NODES = 100000
N_EDGES = 1600000
U = 3.5
D = 2.0

LANES = 16
NC = 2
NS = 16
NW = NC * NS

N_PAD = 100352
SLICE = N_PAD // NS
E_PER_W = N_EDGES // NW
CHUNK = 2000
N_CHUNKS = E_PER_W // CHUNK
NBUF = 4
PREF = 2

ROWS = N_PAD // 128


def _h_body(x_ref, h_ref):
    h_ref[...] = 1.0 / (1.0 + jnp.exp(U - D * x_ref[...]))


def _combine_body(x_ref, p_ref, f_ref):
    f_ref[...] = p_ref[:ROWS, :] + p_ref[ROWS:, :] - x_ref[...]


def _edge_body(h_hbm, ei_hbm, w_hbm, z_hbm, out_hbm,
               h_v, src_v, dst_v, w_v, agg_sh,
               sem_h, sem_z, sem_in, sem_sc):
    c = lax.axis_index("c")
    s = lax.axis_index("s")
    wid = c * NS + s
    base = wid * E_PER_W
    sl = pl.ds(s * SLICE, SLICE)

    zd = pltpu.async_copy(z_hbm.at[sl], agg_sh.at[sl], sem_z)
    hd = pltpu.async_copy(h_hbm.at[pl.ds(0, N_NODES)], h_v, sem_h)

    def issue_inputs(j):
        b = j % NBUF
        off = j * CHUNK
        return (
            pltpu.async_copy(ei_hbm.at[0, pl.ds(base + off, CHUNK)], src_v[b], sem_in[b]),
            pltpu.async_copy(ei_hbm.at[1, pl.ds(base + off, CHUNK)], dst_v[b], sem_in[b]),
            pltpu.async_copy(w_hbm.at[pl.ds(base + off, CHUNK)], w_v[b], sem_in[b]),
        )

    in_descs = {j: issue_inputs(j) for j in range(min(PREF, N_CHUNKS))}
    sc_descs = {}

    zd.wait()
    hd.wait()
    plsc.subcore_barrier()

    for j in range(N_CHUNKS):
        b = j % NBUF
        if j + PREF < N_CHUNKS:
            if j + PREF >= NBUF:
                sc_descs.pop(j + PREF - NBUF).wait()
            in_descs[j + PREF] = issue_inputs(j + PREF)
        for d in in_descs.pop(j):
            d.wait()

        def gather_loop(i, _):
            ds16 = pl.ds(i * LANES, LANES)
            w_v[b][ds16] = plsc.load_gather(h_v, [src_v[b][ds16]]) * w_v[b][ds16]
            return 0
        lax.fori_loop(0, CHUNK // LANES, gather_loop, 0, unroll=8)

        sc_descs[j] = pltpu.async_copy(
            w_v[b], agg_sh.at[dst_v[b]], sem_sc[b], add=True)

    for j in sorted(sc_descs):
        sc_descs.pop(j).wait()
    plsc.subcore_barrier()
    pltpu.sync_copy(agg_sh.at[sl], out_hbm.at[pl.ds(wid * SLICE, SLICE)])


_edge_kernel = functools.partial(
    pl.kernel,
    out_type=jax.ShapeDtypeStruct((NC * N_PAD,), jnp.float32),
    mesh=plsc.VectorSubcoreMesh(core_axis_name="c", subcore_axis_name="s"),
    compiler_params=pltpu.CompilerParams(
        needs_layout_passes=False, use_tc_tiling_on_sc=False),
    scratch_types=[
        pltpu.VMEM((N_NODES,), jnp.float32),
        [pltpu.VMEM((CHUNK,), jnp.int32)] * NBUF,
        [pltpu.VMEM((CHUNK,), jnp.int32)] * NBUF,
        [pltpu.VMEM((CHUNK,), jnp.float32)] * NBUF,
        pltpu.VMEM_SHARED((N_PAD,), jnp.float32),
        pltpu.SemaphoreType.DMA,
        pltpu.SemaphoreType.DMA,
        [pltpu.SemaphoreType.DMA] * NBUF,
        [pltpu.SemaphoreType.DMA] * NBUF,
    ],
)(_edge_body)


@jax.jit
def kernel(t, x, edge_index, edge_weight):
    del t
    x_pad = jnp.pad(x, ((0, N_PAD - N_NODES), (0, 0))).reshape(ROWS, 128)

    h2d = pl.pallas_call(
        _h_body,
        out_shape=jax.ShapeDtypeStruct((ROWS, 128), jnp.float32),
    )(x_pad)
    h = h2d.reshape(N_PAD)

    zeros = jnp.zeros((N_PAD,), jnp.float32)
    partials = _edge_kernel(h, edge_index.astype(jnp.int32), edge_weight, zeros)

    f2d = pl.pallas_call(
        _combine_body,
        out_shape=jax.ShapeDtypeStruct((ROWS, 128), jnp.float32),
    )(x_pad, partials.reshape(2 * ROWS, 128))
    return f2d.reshape(N_PAD)[:N_NODES].reshape(N_NODES, 1)
